# Initial kernel scaffold; baseline (speedup 1.0000x reference)
import jax
import jax.numpy as jnp
from jax.experimental import pallas as pl


def kernel(image, ddf):
    raise NotImplementedError("write your pallas kernel here")


# speedup vs baseline: 2.8027x; 2.8027x over previous
from functools import partial

import jax
import jax.numpy as jnp
from jax.experimental import pallas as pl
from jax.experimental.pallas import tpu as pltpu

_VMEM_LIMIT = 64 * 1024 * 1024


def _warp_body(img_ref, ddf_ref, out_ref, *, D, H, W, tile_n):
    HW = H * W
    CD = img_ref.shape[1]
    C = CD // D
    t = pl.program_id(1)

    ddf = ddf_ref[0]
    n = t * tile_n + jax.lax.broadcasted_iota(jnp.int32, (1, tile_n), 1)
    d_idx = n // HW
    rem = n - d_idx * HW
    h_idx = rem // W
    w_idx = rem - h_idx * W

    z = jnp.clip(d_idx.astype(jnp.float32) + ddf[0:1, :], 0.0, float(D - 1))
    y = jnp.clip(h_idx.astype(jnp.float32) + ddf[1:2, :], 0.0, float(H - 1))
    x = jnp.clip(w_idx.astype(jnp.float32) + ddf[2:3, :], 0.0, float(W - 1))

    z0f = jnp.floor(z)
    y0f = jnp.floor(y)
    x0f = jnp.floor(x)
    fz = z - z0f
    fy = y - y0f
    fx = x - x0f
    z0 = z0f.astype(jnp.int32)
    y0 = y0f.astype(jnp.int32)
    x0 = x0f.astype(jnp.int32)
    z1 = jnp.minimum(z0 + 1, D - 1)
    y1 = jnp.minimum(y0 + 1, H - 1)
    x1 = jnp.minimum(x0 + 1, W - 1)

    wy0 = 1.0 - fy
    wx0 = 1.0 - fx

    s2 = jax.lax.broadcasted_iota(jnp.int32, (HW, tile_n), 0)
    slab = jnp.where(s2 == y0 * W + x0, wy0 * wx0, 0.0)
    slab = slab + jnp.where(s2 == y0 * W + x1, wy0 * fx, 0.0)
    slab = slab + jnp.where(s2 == y1 * W + x0, fy * wx0, 0.0)
    slab = slab + jnp.where(s2 == y1 * W + x1, fy * fx, 0.0)

    lhs = img_ref[0].astype(jnp.bfloat16)
    a = jnp.dot(lhs, slab.astype(jnp.bfloat16),
                preferred_element_type=jnp.float32)
    a3 = a.reshape(C, D, tile_n)

    zi = jax.lax.broadcasted_iota(jnp.int32, (D, tile_n), 0)
    wz = (jnp.where(zi == z0, 1.0 - fz, 0.0)
          + jnp.where(zi == z1, fz, 0.0))
    out = jnp.sum(a3 * wz[None, :, :], axis=1)
    out_ref[0] = out.astype(out_ref.dtype)


def kernel(image, ddf):
    B, C, D, H, W = image.shape
    HW = H * W
    N = D * HW
    tile_n = min(N, 1024)

    img2 = image.reshape(B, C * D, HW)
    ddf2 = ddf.reshape(B, 3, N).astype(jnp.float32)

    body = partial(_warp_body, D=D, H=H, W=W, tile_n=tile_n)
    out = pl.pallas_call(
        body,
        out_shape=jax.ShapeDtypeStruct((B, C, N), image.dtype),
        grid_spec=pltpu.PrefetchScalarGridSpec(
            num_scalar_prefetch=0,
            grid=(B, N // tile_n),
            in_specs=[
                pl.BlockSpec((1, C * D, HW), lambda b, t: (b, 0, 0)),
                pl.BlockSpec((1, 3, tile_n), lambda b, t: (b, 0, t)),
            ],
            out_specs=pl.BlockSpec((1, C, tile_n), lambda b, t: (b, 0, t)),
        ),
        compiler_params=pltpu.CompilerParams(
            dimension_semantics=("parallel", "parallel"),
            vmem_limit_bytes=_VMEM_LIMIT,
        ),
    )(img2, ddf2)
    return out.reshape(B, C, D, H, W)


# speedup vs baseline: 4.6607x; 1.6630x over previous
from functools import partial

import jax
import jax.numpy as jnp
from jax.experimental import pallas as pl
from jax.experimental.pallas import tpu as pltpu

_VMEM_LIMIT = 64 * 1024 * 1024


def _warp_body(img_ref, ddf_ref, out_ref, *, D, H, W, tile_n):
    HW = H * W
    C = img_ref.shape[1]
    t = pl.program_id(1)

    ddf = ddf_ref[0]
    n = t * tile_n + jax.lax.broadcasted_iota(jnp.int32, (1, tile_n), 1)
    d_idx = n // HW
    rem = n - d_idx * HW
    h_idx = rem // W
    w_idx = rem - h_idx * W

    z = jnp.clip(d_idx.astype(jnp.float32) + ddf[0:1, :], 0.0, float(D - 1))
    y = jnp.clip(h_idx.astype(jnp.float32) + ddf[1:2, :], 0.0, float(H - 1))
    x = jnp.clip(w_idx.astype(jnp.float32) + ddf[2:3, :], 0.0, float(W - 1))

    z0f = jnp.floor(z)
    y0f = jnp.floor(y)
    x0f = jnp.floor(x)
    fz = z - z0f
    fy = y - y0f
    fx = x - x0f
    z0 = z0f.astype(jnp.int32)
    y0 = y0f.astype(jnp.int32)
    x0 = x0f.astype(jnp.int32)
    z1 = jnp.minimum(z0 + 1, D - 1)
    y1 = jnp.minimum(y0 + 1, H - 1)
    x1 = jnp.minimum(x0 + 1, W - 1)

    wy0 = 1.0 - fy
    wx0 = 1.0 - fx

    s2 = jax.lax.broadcasted_iota(jnp.int32, (HW, tile_n), 0)
    slab = jnp.where(s2 == y0 * W + x0, wy0 * wx0, 0.0)
    slab = slab + jnp.where(s2 == y0 * W + x1, wy0 * fx, 0.0)
    slab = slab + jnp.where(s2 == y1 * W + x0, fy * wx0, 0.0)
    slab = slab + jnp.where(s2 == y1 * W + x1, fy * fx, 0.0)

    lhs = img_ref[0].reshape(C * D, HW).astype(jnp.bfloat16)
    a = jnp.dot(lhs, slab.astype(jnp.bfloat16),
                preferred_element_type=jnp.float32)
    a3 = a.reshape(C, D, tile_n)

    zi = jax.lax.broadcasted_iota(jnp.int32, (D, tile_n), 0)
    wz = (jnp.where(zi == z0, 1.0 - fz, 0.0)
          + jnp.where(zi == z1, fz, 0.0))
    out = jnp.sum(a3 * wz[None, :, :], axis=1)
    out_ref[0] = out.astype(out_ref.dtype)


def kernel(image, ddf):
    B, C, D, H, W = image.shape
    HW = H * W
    N = D * HW
    tile_n = min(N, 1024)

    img2 = image.reshape(B, C, D, HW)
    ddf2 = ddf.reshape(B, 3, N).astype(jnp.float32)

    body = partial(_warp_body, D=D, H=H, W=W, tile_n=tile_n)
    out = pl.pallas_call(
        body,
        out_shape=jax.ShapeDtypeStruct((B, C, N), image.dtype),
        grid_spec=pltpu.PrefetchScalarGridSpec(
            num_scalar_prefetch=0,
            grid=(B, N // tile_n),
            in_specs=[
                pl.BlockSpec((1, C, D, HW), lambda b, t: (b, 0, 0, 0)),
                pl.BlockSpec((1, 3, tile_n), lambda b, t: (b, 0, t)),
            ],
            out_specs=pl.BlockSpec((1, C, tile_n), lambda b, t: (b, 0, t)),
        ),
        compiler_params=pltpu.CompilerParams(
            dimension_semantics=("parallel", "parallel"),
            vmem_limit_bytes=_VMEM_LIMIT,
        ),
    )(img2, ddf2)
    return out.reshape(B, C, D, H, W)


# speedup vs baseline: 5.2156x; 1.1191x over previous
from functools import partial

import jax
import jax.numpy as jnp
from jax.experimental import pallas as pl
from jax.experimental.pallas import tpu as pltpu

_VMEM_LIMIT = 64 * 1024 * 1024


def _warp_body(img_ref, ddf_ref, out_ref, *, D, H, W, tile_n):
    HW = H * W
    C = img_ref.shape[1]
    t = pl.program_id(1)

    ddf = ddf_ref[0]
    n = t * tile_n + jax.lax.broadcasted_iota(jnp.int32, (1, tile_n), 1)
    d_idx = n // HW
    rem = n - d_idx * HW
    h_idx = rem // W
    w_idx = rem - h_idx * W

    z = jnp.clip(d_idx.astype(jnp.float32) + ddf[0:1, :], 0.0, float(D - 1))
    y = jnp.clip(h_idx.astype(jnp.float32) + ddf[1:2, :], 0.0, float(H - 1))
    x = jnp.clip(w_idx.astype(jnp.float32) + ddf[2:3, :], 0.0, float(W - 1))

    z0f = jnp.floor(z)
    y0f = jnp.floor(y)
    x0f = jnp.floor(x)
    fz = z - z0f
    fy = y - y0f
    fx = x - x0f
    z0 = z0f.astype(jnp.int32)
    y0 = y0f.astype(jnp.int32)
    x0 = x0f.astype(jnp.int32)
    z1 = jnp.minimum(z0 + 1, D - 1)
    y1 = jnp.minimum(y0 + 1, H - 1)
    x1 = jnp.minimum(x0 + 1, W - 1)

    wy0 = 1.0 - fy
    wx0 = 1.0 - fx

    hh = jax.lax.broadcasted_iota(jnp.int32, (H, tile_n), 0)
    ww = jax.lax.broadcasted_iota(jnp.int32, (W, tile_n), 0)
    wy_f = jnp.where(hh == y0, wy0, 0.0) + jnp.where(hh == y1, fy, 0.0)
    wx_f = jnp.where(ww == x0, wx0, 0.0) + jnp.where(ww == x1, fx, 0.0)
    slab = (wy_f[:, None, :] * wx_f[None, :, :]).reshape(HW, tile_n)

    lhs = img_ref[0].reshape(C * D, HW).astype(jnp.bfloat16)
    a = jnp.dot(lhs, slab.astype(jnp.bfloat16),
                preferred_element_type=jnp.float32)
    a3 = a.reshape(C, D, tile_n)

    zi = jax.lax.broadcasted_iota(jnp.int32, (D, tile_n), 0)
    wz = (jnp.where(zi == z0, 1.0 - fz, 0.0)
          + jnp.where(zi == z1, fz, 0.0))
    out = jnp.sum(a3 * wz[None, :, :], axis=1)
    out_ref[0] = out.astype(out_ref.dtype)


def kernel(image, ddf):
    B, C, D, H, W = image.shape
    HW = H * W
    N = D * HW
    tile_n = min(N, 1024)

    img2 = image.reshape(B, C, D, HW)
    ddf2 = ddf.reshape(B, 3, N).astype(jnp.float32)

    body = partial(_warp_body, D=D, H=H, W=W, tile_n=tile_n)
    out = pl.pallas_call(
        body,
        out_shape=jax.ShapeDtypeStruct((B, C, N), image.dtype),
        grid_spec=pltpu.PrefetchScalarGridSpec(
            num_scalar_prefetch=0,
            grid=(B, N // tile_n),
            in_specs=[
                pl.BlockSpec((1, C, D, HW), lambda b, t: (b, 0, 0, 0)),
                pl.BlockSpec((1, 3, tile_n), lambda b, t: (b, 0, t)),
            ],
            out_specs=pl.BlockSpec((1, C, tile_n), lambda b, t: (b, 0, t)),
        ),
        compiler_params=pltpu.CompilerParams(
            dimension_semantics=("parallel", "parallel"),
            vmem_limit_bytes=_VMEM_LIMIT,
        ),
    )(img2, ddf2)
    return out.reshape(B, C, D, H, W)


# speedup vs baseline: 5.2220x; 1.0012x over previous
from functools import partial

import jax
import jax.numpy as jnp
from jax.experimental import pallas as pl
from jax.experimental.pallas import tpu as pltpu

_VMEM_LIMIT = 64 * 1024 * 1024


def _warp_body(img_ref, ddf_ref, out_ref, *, D, H, W, tile_n):
    HW = H * W
    C = img_ref.shape[1]
    t = pl.program_id(1)

    ddf = ddf_ref[0]
    n = t * tile_n + jax.lax.broadcasted_iota(jnp.int32, (1, tile_n), 1)
    d_idx = n // HW
    rem = n - d_idx * HW
    h_idx = rem // W
    w_idx = rem - h_idx * W

    z = jnp.clip(d_idx.astype(jnp.float32) + ddf[0:1, :], 0.0, float(D - 1))
    y = jnp.clip(h_idx.astype(jnp.float32) + ddf[1:2, :], 0.0, float(H - 1))
    x = jnp.clip(w_idx.astype(jnp.float32) + ddf[2:3, :], 0.0, float(W - 1))

    z0f = jnp.floor(z)
    y0f = jnp.floor(y)
    x0f = jnp.floor(x)
    fz = z - z0f
    fy = y - y0f
    fx = x - x0f
    z0 = z0f.astype(jnp.int32)
    y0 = y0f.astype(jnp.int32)
    x0 = x0f.astype(jnp.int32)
    z1 = jnp.minimum(z0 + 1, D - 1)
    y1 = jnp.minimum(y0 + 1, H - 1)
    x1 = jnp.minimum(x0 + 1, W - 1)

    wy0 = 1.0 - fy
    wx0 = 1.0 - fx

    hh = jax.lax.broadcasted_iota(jnp.int32, (H, tile_n), 0)
    ww = jax.lax.broadcasted_iota(jnp.int32, (W, tile_n), 0)
    wy_f = jnp.where(hh == y0, wy0, 0.0) + jnp.where(hh == y1, fy, 0.0)
    wx_f = jnp.where(ww == x0, wx0, 0.0) + jnp.where(ww == x1, fx, 0.0)
    slab = (wy_f[:, None, :] * wx_f[None, :, :]).reshape(HW, tile_n)

    lhs = img_ref[0].reshape(C * D, HW).astype(jnp.bfloat16)
    a = jnp.dot(lhs, slab.astype(jnp.bfloat16),
                preferred_element_type=jnp.float32)
    a3 = a.reshape(C, D, tile_n)

    zi = jax.lax.broadcasted_iota(jnp.int32, (D, tile_n), 0)
    wz = (jnp.where(zi == z0, 1.0 - fz, 0.0)
          + jnp.where(zi == z1, fz, 0.0))
    out = jnp.sum(a3 * wz[None, :, :], axis=1)
    out_ref[0] = out.astype(out_ref.dtype)


def kernel(image, ddf):
    B, C, D, H, W = image.shape
    HW = H * W
    N = D * HW
    tile_n = min(N, 2048)

    img2 = image.reshape(B, C, D, HW)
    ddf2 = ddf.reshape(B, 3, N).astype(jnp.float32)

    body = partial(_warp_body, D=D, H=H, W=W, tile_n=tile_n)
    out = pl.pallas_call(
        body,
        out_shape=jax.ShapeDtypeStruct((B, C, N), image.dtype),
        grid_spec=pltpu.PrefetchScalarGridSpec(
            num_scalar_prefetch=0,
            grid=(B, N // tile_n),
            in_specs=[
                pl.BlockSpec((1, C, D, HW), lambda b, t: (b, 0, 0, 0)),
                pl.BlockSpec((1, 3, tile_n), lambda b, t: (b, 0, t)),
            ],
            out_specs=pl.BlockSpec((1, C, tile_n), lambda b, t: (b, 0, t)),
        ),
        compiler_params=pltpu.CompilerParams(
            dimension_semantics=("parallel", "parallel"),
            vmem_limit_bytes=_VMEM_LIMIT,
        ),
    )(img2, ddf2)
    return out.reshape(B, C, D, H, W)
